# baseline (speedup 1.0000x reference)
import jax
import jax.numpy as jnp
from jax import lax
from jax.experimental import pallas as pl
from jax.experimental.pallas import tpu as pltpu
from jax.experimental.pallas import tpu_sc as plsc

N = 10000
E = 320000
F = 128
FH = 64
EMB = 64
N_IDX = 2048

NC = 2
NS = 16
NW = NC * NS
ET = E // NS
K = 128
NB = 158
EP = NB * K
RPT = N // NS
ZR = 125

_SC_PARAMS = pltpu.CompilerParams(use_tc_tiling_on_sc=False,
                                  needs_layout_passes=False)


def _edge_body(h2_hbm, edges_hbm, ss_hbm, sn_hbm, accf_out, den_out,
               ss_v, sn_v, eij2, eibC, ejbC, wC, rows2, srows2,
               eiS, wS, zb, zd, acc_sh, den_sh,
               gsem0, gsem1, ssem0, ssem1, esem0, esem1):
    cid = lax.axis_index("c")
    sid = lax.axis_index("s")
    jbase = (cid * N).astype(jnp.int32)

    zero16 = jnp.zeros((16,), jnp.float32)
    def _zb(r, _):
        for c in range(FH // 16):
            zb[r, pl.ds(c * 16, 16)] = zero16
        return 0
    lax.fori_loop(0, ZR, _zb, 0)
    def _zd(i, _):
        zd[pl.ds(i * 16, 16)] = zero16
        return 0
    lax.fori_loop(0, 63, _zd, 0)
    for t in range(RPT // ZR):
        pltpu.sync_copy(zb, acc_sh.at[pl.ds(sid * RPT + t * ZR, ZR)])
    @pl.when(sid < 10)
    def _():
        pltpu.sync_copy(zd.at[pl.ds(0, 1000)],
                        den_sh.at[pl.ds(sid * 1000, 1000)])

    pltpu.sync_copy(ss_hbm, ss_v)
    pltpu.sync_copy(sn_hbm, sn_v)

    ninf = jnp.full((16,), -jnp.inf, jnp.float32)
    def _mx(i, carry):
        ms, mn = carry
        ms = jnp.maximum(ms, ss_v[pl.ds(i * 16, 16)])
        mn = jnp.maximum(mn, sn_v[pl.ds(i * 16, 16)])
        return ms, mn
    ms, mn = lax.fori_loop(0, N // 16, _mx, (ninf, ninf))
    sv = jnp.max(ms) + jnp.max(mn)
    mval = jnp.where(sv >= 0.0, sv, 0.2 * sv)

    plsc.subcore_barrier()

    lane16 = lax.iota(jnp.int32, 16)

    def _wcomp(b, d):
        base = b * K
        for g in range(K // 16):
            o = g * 16
            eiv = eij2[d, 0, pl.ds(o, 16)]
            ejv = eij2[d, 1, pl.ds(o, 16)]
            eibC[d, pl.ds(o, 16)] = eiv
            ejbC[d, pl.ds(o, 16)] = ejv + jbase
            e = (plsc.load_gather(ss_v, [eiv]) +
                 plsc.load_gather(sn_v, [ejv]))
            e = jnp.where(e >= 0.0, e, 0.2 * e)
            w = jnp.exp(e - mval)
            wC[d, pl.ds(o, 16)] = jnp.where(base + o + lane16 < ET, w, 0.0)

    def _gather_start(d, sem):
        pltpu.make_async_copy(h2_hbm.at[ejbC.at[d]], rows2.at[d],
                              sem).start()

    def _scat_descs(d, sem):
        return (pltpu.make_async_copy(srows2.at[d], acc_sh.at[eiS.at[d]],
                                      sem),
                pltpu.make_async_copy(wS.at[d], den_sh.at[eiS.at[d]], sem))

    esems = (esem0, esem1)
    gsems = (gsem0, gsem1)
    ssems = (ssem0, ssem1)

    def _eij_start(bb, dd):
        pltpu.make_async_copy(edges_hbm.at[sid, bb], eij2.at[dd],
                              esems[dd]).start()

    def _eij_wait(dd):
        pltpu.make_async_copy(edges_hbm.at[sid, 0], eij2.at[dd],
                              esems[dd]).wait()

    pltpu.sync_copy(edges_hbm.at[sid, 0], eij2.at[0])
    _eij_start(1, 1)
    _wcomp(0, 0)
    _gather_start(0, gsem0)

    def _pair(p, _):
        for d in (0, 1):
            b = 2 * p + d
            d1 = 1 - d
            @pl.when(jnp.logical_or(d == 0, p < NB // 2 - 1))
            def _():
                _eij_wait(d1)
                _wcomp(b + 1, d1)
                _gather_start(d1, gsems[d1])
            @pl.when(p < NB // 2 - 1)
            def _():
                _eij_start(b + 2, d)
            @pl.when(p >= 1)
            def _():
                fd, dd = _scat_descs(d, ssems[d])
                fd.wait()
                dd.wait()
            for g in range(K // 16):
                o = g * 16
                eiS[d, pl.ds(o, 16)] = eibC[d, pl.ds(o, 16)]
                wS[d, pl.ds(o, 16)] = wC[d, pl.ds(o, 16)]
            pltpu.make_async_copy(h2_hbm.at[ejbC.at[d]], rows2.at[d],
                                  gsems[d]).wait()
            def _grp(g, _):
                o = g * 16
                w16 = wC[d, pl.ds(o, 16)]
                for u in range(16):
                    wsp = lax.gather(
                        w16,
                        jnp.full((16, 1), u, jnp.int32),
                        lax.GatherDimensionNumbers(
                            offset_dims=(), collapsed_slice_dims=(0,),
                            start_index_map=(0,)),
                        (1,),
                        mode=lax.GatherScatterMode.PROMISE_IN_BOUNDS)
                    r = o + u
                    for c in range(FH // 16):
                        co = c * 16
                        srows2[d, r, pl.ds(co, 16)] = \
                            wsp * rows2[d, r, pl.ds(co, 16)]
                return 0
            lax.fori_loop(0, K // 16, _grp, 0)
            fd, dd = _scat_descs(d, ssems[d])
            fd.start(add=True)
            dd.start(add=True)
        return 0

    lax.fori_loop(0, NB // 2, _pair, 0)
    for d in (0, 1):
        fd, dd = _scat_descs(d, ssems[d])
        fd.wait()
        dd.wait()

    plsc.subcore_barrier()
    pltpu.sync_copy(acc_sh.at[pl.ds(sid * RPT, RPT)],
                    accf_out.at[cid, pl.ds(sid * RPT, RPT)])
    @pl.when(sid < 10)
    def _():
        pltpu.sync_copy(den_sh.at[pl.ds(sid * 1000, 1000)],
                        den_out.at[cid, pl.ds(sid * 1000, 1000)])


_edge_call = pl.kernel(
    _edge_body,
    out_type=[jax.ShapeDtypeStruct((NC, N, FH), jnp.float32),
              jax.ShapeDtypeStruct((NC, N), jnp.float32)],
    mesh=plsc.VectorSubcoreMesh(core_axis_name="c", subcore_axis_name="s",
                                num_cores=NC, num_subcores=NS),
    scratch_types=[
        pltpu.VMEM((N,), jnp.float32),
        pltpu.VMEM((N,), jnp.float32),
        pltpu.VMEM((2, 2, K), jnp.int32),
        pltpu.VMEM((2, K), jnp.int32),
        pltpu.VMEM((2, K), jnp.int32),
        pltpu.VMEM((2, K), jnp.float32),
        pltpu.VMEM((2, K, FH), jnp.float32),
        pltpu.VMEM((2, K, FH), jnp.float32),
        pltpu.VMEM((2, K), jnp.int32),
        pltpu.VMEM((2, K), jnp.float32),
        pltpu.VMEM((ZR, FH), jnp.float32),
        pltpu.VMEM((1008,), jnp.float32),
        pltpu.VMEM_SHARED((N, FH), jnp.float32),
        pltpu.VMEM_SHARED((N,), jnp.float32),
        pltpu.SemaphoreType.DMA,
        pltpu.SemaphoreType.DMA,
        pltpu.SemaphoreType.DMA,
        pltpu.SemaphoreType.DMA,
        pltpu.SemaphoreType.DMA,
        pltpu.SemaphoreType.DMA,
    ],
    compiler_params=_SC_PARAMS,
    name="gat_edge_sc",
)


def _gather_body(full_hbm, idx_hbm, out_hbm, idx_v, rows_v, sem):
    wid = lax.axis_index("c") * NS + lax.axis_index("s")
    base = wid * (N_IDX // NW)
    pltpu.sync_copy(idx_hbm.at[pl.ds(base, N_IDX // NW)], idx_v)
    pltpu.async_copy(full_hbm.at[idx_v], rows_v, sem).wait()
    pltpu.sync_copy(rows_v, out_hbm.at[pl.ds(base, N_IDX // NW)])


_gather_call = pl.kernel(
    _gather_body,
    out_type=jax.ShapeDtypeStruct((N_IDX, EMB), jnp.float32),
    mesh=plsc.VectorSubcoreMesh(core_axis_name="c", subcore_axis_name="s",
                                num_cores=NC, num_subcores=NS),
    scratch_types=[
        pltpu.VMEM((N_IDX // NW,), jnp.int32),
        pltpu.VMEM((N_IDX // NW, EMB), jnp.float32),
        pltpu.SemaphoreType.DMA,
    ],
    compiler_params=_SC_PARAMS,
    name="gat_idx_gather_sc",
)


ROWS_B = 2000


def _split_h(h, h2_ref):
    h2_ref[0] = h[:, :FH]
    h2_ref[1] = h[:, FH:]


def _tc0_body(x_ref, w_ref, a_ref, h2_ref, s2_ref):
    h = jnp.dot(x_ref[...], w_ref[...], preferred_element_type=jnp.float32)
    _split_h(h, h2_ref)
    s2_ref[...] = jnp.dot(h, a_ref[...], preferred_element_type=jnp.float32)


def _tc0(x, w, a2):
    return pl.pallas_call(
        _tc0_body,
        grid=(N // ROWS_B,),
        in_specs=[
            pl.BlockSpec((ROWS_B, F), lambda i: (i, 0)),
            pl.BlockSpec((F, F), lambda i: (0, 0)),
            pl.BlockSpec((F, 2), lambda i: (0, 0)),
        ],
        out_specs=[
            pl.BlockSpec((NC, ROWS_B, FH), lambda i: (0, i, 0)),
            pl.BlockSpec((ROWS_B, 2), lambda i: (i, 0)),
        ],
        out_shape=[
            jax.ShapeDtypeStruct((NC, N, FH), jnp.float32),
            jax.ShapeDtypeStruct((N, 2), jnp.float32),
        ],
        name="gat_dense0_tc",
    )(x, w, a2)


def _finish(acc_ref, den_ref, b_ref, g_ref, be_ref, mu_ref, va_ref):
    num = jnp.concatenate([acc_ref[0, :, :FH], acc_ref[1, :, :FH]], axis=1)
    den = den_ref[...]
    y = jnp.where(den > 0.0, num / jnp.where(den > 0.0, den, 1.0), 0.0)
    y = y + b_ref[...]
    y = g_ref[...] * (y - mu_ref[...]) * lax.rsqrt(va_ref[...] + 1e-5) \
        + be_ref[...]
    return jnp.maximum(y, 0.0)


def _tc1_body(acc_ref, den_ref, b_ref, g_ref, be_ref, mu_ref, va_ref,
              w_ref, a_ref, h2_ref, s2_ref):
    x = _finish(acc_ref, den_ref, b_ref, g_ref, be_ref, mu_ref, va_ref)
    h = jnp.dot(x, w_ref[...], preferred_element_type=jnp.float32)
    _split_h(h, h2_ref)
    s2_ref[...] = jnp.dot(h, a_ref[...], preferred_element_type=jnp.float32)


def _tc1(acc, den, b, g, be, mu, va, w, a2):
    vec = [pl.BlockSpec((1, F), lambda i: (0, 0))] * 5
    return pl.pallas_call(
        _tc1_body,
        grid=(N // ROWS_B,),
        in_specs=[pl.BlockSpec((NC, ROWS_B, FH), lambda i: (0, i, 0)),
                  pl.BlockSpec((ROWS_B, 1), lambda i: (i, 0))] + vec
        + [
            pl.BlockSpec((F, F), lambda i: (0, 0)),
            pl.BlockSpec((F, 2), lambda i: (0, 0)),
        ],
        out_specs=[
            pl.BlockSpec((NC, ROWS_B, FH), lambda i: (0, i, 0)),
            pl.BlockSpec((ROWS_B, 2), lambda i: (i, 0)),
        ],
        out_shape=[
            jax.ShapeDtypeStruct((NC, N, FH), jnp.float32),
            jax.ShapeDtypeStruct((N, 2), jnp.float32),
        ],
        name="gat_dense1_tc",
    )(acc, den, b, g, be, mu, va, w, a2)


def _tc2_body(acc_ref, den_ref, b_ref, g_ref, be_ref, mu_ref, va_ref,
              wd_ref, bd_ref, out_ref):
    x = _finish(acc_ref, den_ref, b_ref, g_ref, be_ref, mu_ref, va_ref)
    out_ref[...] = jnp.dot(x, wd_ref[...],
                           preferred_element_type=jnp.float32) + bd_ref[...]


def _tc2(acc, den, b, g, be, mu, va, wd, bd):
    vec = [pl.BlockSpec((1, F), lambda i: (0, 0))] * 5
    return pl.pallas_call(
        _tc2_body,
        grid=(N // ROWS_B,),
        in_specs=[pl.BlockSpec((NC, ROWS_B, FH), lambda i: (0, i, 0)),
                  pl.BlockSpec((ROWS_B, 1), lambda i: (i, 0))] + vec
        + [
            pl.BlockSpec((F, EMB), lambda i: (0, 0)),
            pl.BlockSpec((1, EMB), lambda i: (0, 0)),
        ],
        out_specs=pl.BlockSpec((ROWS_B, EMB), lambda i: (i, 0)),
        out_shape=jax.ShapeDtypeStruct((N, EMB), jnp.float32),
        name="gat_dense2_tc",
    )(acc, den, b, g, be, mu, va, wd, bd)


def kernel(features, edge_index, idx, W0, a1_0, a2_0, b0, gamma0, beta0,
           mean0, var0, W1, a1_1, a2_1, b1, gamma1, beta1, mean1, var1,
           Wd, bd):
    epad = jnp.pad(edge_index.reshape(2, NS, ET),
                   ((0, 0), (0, 0), (0, EP - ET)))
    edges4 = jnp.stack([epad[0].reshape(NS, NB, K),
                        epad[1].reshape(NS, NB, K)], axis=2)
    A0 = jnp.stack([a1_0, a2_0], axis=1)
    A1 = jnp.stack([a1_1, a2_1], axis=1)
    r = lambda v: v.reshape(1, F)

    h20, s20 = _tc0(features, W0, A0)
    acc0, den0 = _edge_call(h20.reshape(NC * N, FH), edges4,
                            s20[:, 0], s20[:, 1])
    h21, s21 = _tc1(acc0, den0[0].reshape(N, 1), r(b0), r(gamma0), r(beta0), r(mean0),
                    r(var0), W1, A1)
    acc1, den1 = _edge_call(h21.reshape(NC * N, FH), edges4,
                            s21[:, 0], s21[:, 1])
    full = _tc2(acc1, den1[0].reshape(N, 1), r(b1), r(gamma1), r(beta1), r(mean1), r(var1),
                Wd, bd.reshape(1, EMB))
    return _gather_call(full, idx)


# speedup vs baseline: 1.1746x; 1.1746x over previous
import jax
import jax.numpy as jnp
from jax import lax
from jax.experimental import pallas as pl
from jax.experimental.pallas import tpu as pltpu
from jax.experimental.pallas import tpu_sc as plsc

N = 10000
E = 320000
F = 128
FH = 64
EMB = 64
N_IDX = 2048

NC = 2
NS = 16
NW = NC * NS
ET = E // NS
K = 80
NB = ET // K
RPT = N // NS
ZR = 125

_SC_PARAMS = pltpu.CompilerParams(use_tc_tiling_on_sc=False,
                                  needs_layout_passes=False)


def _edge_body(h2_hbm, ei_hbm, ej_hbm, ss_hbm, sn_hbm, accf_out, den_out,
               ss_v, sn_v, ei_v, ej_v, eibC, ejbC, wC, rows2, srows2,
               eiS, wS, zb, zd, acc_sh, den_sh,
               gsem0, gsem1, ssem0, ssem1):
    cid = lax.axis_index("c")
    sid = lax.axis_index("s")
    jbase = (cid * N).astype(jnp.int32)

    zero16 = jnp.zeros((16,), jnp.float32)
    def _zb(r, _):
        for c in range(FH // 16):
            zb[r, pl.ds(c * 16, 16)] = zero16
        return 0
    lax.fori_loop(0, ZR, _zb, 0)
    def _zd(i, _):
        zd[pl.ds(i * 16, 16)] = zero16
        return 0
    lax.fori_loop(0, 63, _zd, 0)
    for t in range(RPT // ZR):
        pltpu.sync_copy(zb, acc_sh.at[pl.ds(sid * RPT + t * ZR, ZR)])
    @pl.when(sid < 10)
    def _():
        pltpu.sync_copy(zd.at[pl.ds(0, 1000)],
                        den_sh.at[pl.ds(sid * 1000, 1000)])

    pltpu.sync_copy(ss_hbm, ss_v)
    pltpu.sync_copy(sn_hbm, sn_v)
    pltpu.sync_copy(ei_hbm.at[sid], ei_v)
    pltpu.sync_copy(ej_hbm.at[sid], ej_v)

    ninf = jnp.full((16,), -jnp.inf, jnp.float32)
    def _mx(i, carry):
        ms, mn = carry
        ms = jnp.maximum(ms, ss_v[pl.ds(i * 16, 16)])
        mn = jnp.maximum(mn, sn_v[pl.ds(i * 16, 16)])
        return ms, mn
    ms, mn = lax.fori_loop(0, N // 16, _mx, (ninf, ninf))
    sv = jnp.max(ms) + jnp.max(mn)
    mval = jnp.where(sv >= 0.0, sv, 0.2 * sv)

    plsc.subcore_barrier()

    def _wcomp(b, d):
        for g in range(K // 16):
            o = g * 16
            eiv = ei_v[b, pl.ds(o, 16)]
            ejv = ej_v[b, pl.ds(o, 16)]
            eibC[d, pl.ds(o, 16)] = eiv
            ejbC[d, pl.ds(o, 16)] = ejv + jbase
            e = (plsc.load_gather(ss_v, [eiv]) +
                 plsc.load_gather(sn_v, [ejv]))
            e = jnp.where(e >= 0.0, e, 0.2 * e)
            wC[d, pl.ds(o, 16)] = jnp.exp(e - mval)

    def _gather_start(d, sem):
        pltpu.make_async_copy(h2_hbm.at[ejbC.at[d]], rows2.at[d],
                              sem).start()

    def _scat_descs(d, sem):
        return (pltpu.make_async_copy(srows2.at[d], acc_sh.at[eiS.at[d]],
                                      sem),
                pltpu.make_async_copy(wS.at[d], den_sh.at[eiS.at[d]], sem))

    _wcomp(0, 0)
    _gather_start(0, gsem0)

    gsems = (gsem0, gsem1)
    ssems = (ssem0, ssem1)

    def _pair(p, _):
        for d in (0, 1):
            b = 2 * p + d
            d1 = 1 - d
            @pl.when(jnp.logical_or(d == 0, p < NB // 2 - 1))
            def _():
                _wcomp(b + 1, d1)
                _gather_start(d1, gsems[d1])
            @pl.when(p >= 1)
            def _():
                fd, dd = _scat_descs(d, ssems[d])
                fd.wait()
                dd.wait()
            for g in range(K // 16):
                o = g * 16
                eiS[d, pl.ds(o, 16)] = eibC[d, pl.ds(o, 16)]
                wS[d, pl.ds(o, 16)] = wC[d, pl.ds(o, 16)]
            pltpu.make_async_copy(h2_hbm.at[ejbC.at[d]], rows2.at[d],
                                  gsems[d]).wait()
            def _grp(g, _):
                o = g * 16
                w16 = wC[d, pl.ds(o, 16)]
                for u in range(16):
                    wsp = lax.gather(
                        w16,
                        jnp.full((16, 1), u, jnp.int32),
                        lax.GatherDimensionNumbers(
                            offset_dims=(), collapsed_slice_dims=(0,),
                            start_index_map=(0,)),
                        (1,),
                        mode=lax.GatherScatterMode.PROMISE_IN_BOUNDS)
                    r = o + u
                    for c in range(FH // 16):
                        co = c * 16
                        srows2[d, r, pl.ds(co, 16)] = \
                            wsp * rows2[d, r, pl.ds(co, 16)]
                return 0
            lax.fori_loop(0, K // 16, _grp, 0)
            fd, dd = _scat_descs(d, ssems[d])
            fd.start(add=True)
            dd.start(add=True)
        return 0

    lax.fori_loop(0, NB // 2, _pair, 0)
    for d in (0, 1):
        fd, dd = _scat_descs(d, ssems[d])
        fd.wait()
        dd.wait()

    plsc.subcore_barrier()
    pltpu.sync_copy(acc_sh.at[pl.ds(sid * RPT, RPT)],
                    accf_out.at[cid, pl.ds(sid * RPT, RPT)])
    @pl.when(sid < 10)
    def _():
        pltpu.sync_copy(den_sh.at[pl.ds(sid * 1000, 1000)],
                        den_out.at[cid, pl.ds(sid * 1000, 1000)])


_edge_call = pl.kernel(
    _edge_body,
    out_type=[jax.ShapeDtypeStruct((NC, N, FH), jnp.float32),
              jax.ShapeDtypeStruct((NC, N), jnp.float32)],
    mesh=plsc.VectorSubcoreMesh(core_axis_name="c", subcore_axis_name="s",
                                num_cores=NC, num_subcores=NS),
    scratch_types=[
        pltpu.VMEM((N,), jnp.float32),
        pltpu.VMEM((N,), jnp.float32),
        pltpu.VMEM((NB, K), jnp.int32),
        pltpu.VMEM((NB, K), jnp.int32),
        pltpu.VMEM((2, K), jnp.int32),
        pltpu.VMEM((2, K), jnp.int32),
        pltpu.VMEM((2, K), jnp.float32),
        pltpu.VMEM((2, K, FH), jnp.float32),
        pltpu.VMEM((2, K, FH), jnp.float32),
        pltpu.VMEM((2, K), jnp.int32),
        pltpu.VMEM((2, K), jnp.float32),
        pltpu.VMEM((ZR, FH), jnp.float32),
        pltpu.VMEM((1008,), jnp.float32),
        pltpu.VMEM_SHARED((N, FH), jnp.float32),
        pltpu.VMEM_SHARED((N,), jnp.float32),
        pltpu.SemaphoreType.DMA,
        pltpu.SemaphoreType.DMA,
        pltpu.SemaphoreType.DMA,
        pltpu.SemaphoreType.DMA,
    ],
    compiler_params=_SC_PARAMS,
    name="gat_edge_sc",
)


def _gather_body(full_hbm, idx_hbm, out_hbm, idx_v, rows_v, sem):
    wid = lax.axis_index("c") * NS + lax.axis_index("s")
    base = wid * (N_IDX // NW)
    pltpu.sync_copy(idx_hbm.at[pl.ds(base, N_IDX // NW)], idx_v)
    pltpu.async_copy(full_hbm.at[idx_v], rows_v, sem).wait()
    pltpu.sync_copy(rows_v, out_hbm.at[pl.ds(base, N_IDX // NW)])


_gather_call = pl.kernel(
    _gather_body,
    out_type=jax.ShapeDtypeStruct((N_IDX, EMB), jnp.float32),
    mesh=plsc.VectorSubcoreMesh(core_axis_name="c", subcore_axis_name="s",
                                num_cores=NC, num_subcores=NS),
    scratch_types=[
        pltpu.VMEM((N_IDX // NW,), jnp.int32),
        pltpu.VMEM((N_IDX // NW, EMB), jnp.float32),
        pltpu.SemaphoreType.DMA,
    ],
    compiler_params=_SC_PARAMS,
    name="gat_idx_gather_sc",
)


ROWS_B = 2000


def _split_h(h, h2_ref):
    h2_ref[0] = h[:, :FH]
    h2_ref[1] = h[:, FH:]


def _tc0_body(x_ref, w_ref, a_ref, h2_ref, s2_ref):
    h = jnp.dot(x_ref[...], w_ref[...], preferred_element_type=jnp.float32)
    _split_h(h, h2_ref)
    s2_ref[...] = jnp.dot(h, a_ref[...], preferred_element_type=jnp.float32)


def _tc0(x, w, a2):
    return pl.pallas_call(
        _tc0_body,
        grid=(N // ROWS_B,),
        in_specs=[
            pl.BlockSpec((ROWS_B, F), lambda i: (i, 0)),
            pl.BlockSpec((F, F), lambda i: (0, 0)),
            pl.BlockSpec((F, 2), lambda i: (0, 0)),
        ],
        out_specs=[
            pl.BlockSpec((NC, ROWS_B, FH), lambda i: (0, i, 0)),
            pl.BlockSpec((ROWS_B, 2), lambda i: (i, 0)),
        ],
        out_shape=[
            jax.ShapeDtypeStruct((NC, N, FH), jnp.float32),
            jax.ShapeDtypeStruct((N, 2), jnp.float32),
        ],
        name="gat_dense0_tc",
    )(x, w, a2)


def _finish(acc_ref, den_ref, b_ref, g_ref, be_ref, mu_ref, va_ref):
    num = jnp.concatenate([acc_ref[0, :, :FH], acc_ref[1, :, :FH]], axis=1)
    den = den_ref[...]
    y = jnp.where(den > 0.0, num / jnp.where(den > 0.0, den, 1.0), 0.0)
    y = y + b_ref[...]
    y = g_ref[...] * (y - mu_ref[...]) * lax.rsqrt(va_ref[...] + 1e-5) \
        + be_ref[...]
    return jnp.maximum(y, 0.0)


def _tc1_body(acc_ref, den_ref, b_ref, g_ref, be_ref, mu_ref, va_ref,
              w_ref, a_ref, h2_ref, s2_ref):
    x = _finish(acc_ref, den_ref, b_ref, g_ref, be_ref, mu_ref, va_ref)
    h = jnp.dot(x, w_ref[...], preferred_element_type=jnp.float32)
    _split_h(h, h2_ref)
    s2_ref[...] = jnp.dot(h, a_ref[...], preferred_element_type=jnp.float32)


def _tc1(acc, den, b, g, be, mu, va, w, a2):
    vec = [pl.BlockSpec((1, F), lambda i: (0, 0))] * 5
    return pl.pallas_call(
        _tc1_body,
        grid=(N // ROWS_B,),
        in_specs=[pl.BlockSpec((NC, ROWS_B, FH), lambda i: (0, i, 0)),
                  pl.BlockSpec((ROWS_B, 1), lambda i: (i, 0))] + vec
        + [
            pl.BlockSpec((F, F), lambda i: (0, 0)),
            pl.BlockSpec((F, 2), lambda i: (0, 0)),
        ],
        out_specs=[
            pl.BlockSpec((NC, ROWS_B, FH), lambda i: (0, i, 0)),
            pl.BlockSpec((ROWS_B, 2), lambda i: (i, 0)),
        ],
        out_shape=[
            jax.ShapeDtypeStruct((NC, N, FH), jnp.float32),
            jax.ShapeDtypeStruct((N, 2), jnp.float32),
        ],
        name="gat_dense1_tc",
    )(acc, den, b, g, be, mu, va, w, a2)


def _tc2_body(acc_ref, den_ref, b_ref, g_ref, be_ref, mu_ref, va_ref,
              wd_ref, bd_ref, out_ref):
    x = _finish(acc_ref, den_ref, b_ref, g_ref, be_ref, mu_ref, va_ref)
    out_ref[...] = jnp.dot(x, wd_ref[...],
                           preferred_element_type=jnp.float32) + bd_ref[...]


def _tc2(acc, den, b, g, be, mu, va, wd, bd):
    vec = [pl.BlockSpec((1, F), lambda i: (0, 0))] * 5
    return pl.pallas_call(
        _tc2_body,
        grid=(N // ROWS_B,),
        in_specs=[pl.BlockSpec((NC, ROWS_B, FH), lambda i: (0, i, 0)),
                  pl.BlockSpec((ROWS_B, 1), lambda i: (i, 0))] + vec
        + [
            pl.BlockSpec((F, EMB), lambda i: (0, 0)),
            pl.BlockSpec((1, EMB), lambda i: (0, 0)),
        ],
        out_specs=pl.BlockSpec((ROWS_B, EMB), lambda i: (i, 0)),
        out_shape=jax.ShapeDtypeStruct((N, EMB), jnp.float32),
        name="gat_dense2_tc",
    )(acc, den, b, g, be, mu, va, wd, bd)


def kernel(features, edge_index, idx, W0, a1_0, a2_0, b0, gamma0, beta0,
           mean0, var0, W1, a1_1, a2_1, b1, gamma1, beta1, mean1, var1,
           Wd, bd):
    ei3 = edge_index[0].reshape(NS, NB, K)
    ej3 = edge_index[1].reshape(NS, NB, K)
    A0 = jnp.stack([a1_0, a2_0], axis=1)
    A1 = jnp.stack([a1_1, a2_1], axis=1)
    r = lambda v: v.reshape(1, F)

    h20, s20 = _tc0(features, W0, A0)
    acc0, den0 = _edge_call(h20.reshape(NC * N, FH), ei3, ej3,
                            s20[:, 0], s20[:, 1])
    h21, s21 = _tc1(acc0, den0[0].reshape(N, 1), r(b0), r(gamma0), r(beta0), r(mean0),
                    r(var0), W1, A1)
    acc1, den1 = _edge_call(h21.reshape(NC * N, FH), ei3, ej3,
                            s21[:, 0], s21[:, 1])
    full = _tc2(acc1, den1[0].reshape(N, 1), r(b1), r(gamma1), r(beta1), r(mean1), r(var1),
                Wd, bd.reshape(1, EMB))
    return _gather_call(full, idx)
